# baseline (speedup 1.0000x reference)
import jax
import jax.numpy as jnp
from jax.experimental import pallas as pl

H = 4
C = 128
D = C // H
NE = 40000
NPASS = 10000


def _lin(x, W, b):
    return x @ W + b


def _ln(x, g, b):
    m = x.mean(-1, keepdims=True)
    v = ((x - m) ** 2).mean(-1, keepdims=True)
    return (x - m) / jnp.sqrt(v + 1e-5) * g + b


def _segment_softmax(logits, seg, num_segments):
    mx = jax.ops.segment_max(logits, seg, num_segments=num_segments)
    mx = jnp.where(jnp.isfinite(mx), mx, 0.0)
    e = jnp.exp(logits - mx[seg])
    s = jax.ops.segment_sum(e, seg, num_segments=num_segments)
    return e / (s[seg] + 1e-16)


def _rel(kv, A):
    return jnp.einsum('nhd,hde->nhe', kv, A)


def _hgt_layer(xe, xp, ei_ee, ei_ep, ei_pe, p):
    sd = jnp.sqrt(jnp.asarray(D, xe.dtype))
    k_e = _lin(xe, p['Wk'][0], p['bk'][0]).reshape(-1, H, D)
    k_p = _lin(xp, p['Wk'][1], p['bk'][1]).reshape(-1, H, D)
    q_e = _lin(xe, p['Wq'][0], p['bq'][0]).reshape(-1, H, D)
    q_p = _lin(xp, p['Wq'][1], p['bq'][1]).reshape(-1, H, D)
    v_e = _lin(xe, p['Wv'][0], p['bv'][0]).reshape(-1, H, D)
    v_p = _lin(xp, p['Wv'][1], p['bv'][1]).reshape(-1, H, D)

    k_ee = _rel(k_e, p['a_rel'][0])[ei_ee[0]]
    v_ee = _rel(v_e, p['m_rel'][0])[ei_ee[0]]
    l_ee = (q_e[ei_ee[1]] * k_ee).sum(-1) * p['p_rel'][0] / sd
    k_pe = _rel(k_p, p['a_rel'][2])[ei_pe[0]]
    v_pe = _rel(v_p, p['m_rel'][2])[ei_pe[0]]
    l_pe = (q_e[ei_pe[1]] * k_pe).sum(-1) * p['p_rel'][2] / sd

    dst_e = jnp.concatenate([ei_ee[1], ei_pe[1]])
    lg_e = jnp.concatenate([l_ee, l_pe], axis=0)
    vv_e = jnp.concatenate([v_ee, v_pe], axis=0)
    a_e = _segment_softmax(lg_e, dst_e, NE)
    agg_e = jax.ops.segment_sum(vv_e * a_e[:, :, None], dst_e, num_segments=NE).reshape(NE, C)

    k_ep = _rel(k_e, p['a_rel'][1])[ei_ep[0]]
    v_ep = _rel(v_e, p['m_rel'][1])[ei_ep[0]]
    l_ep = (q_p[ei_ep[1]] * k_ep).sum(-1) * p['p_rel'][1] / sd
    a_p = _segment_softmax(l_ep, ei_ep[1], NPASS)
    agg_p = jax.ops.segment_sum(v_ep * a_p[:, :, None], ei_ep[1], num_segments=NPASS).reshape(NPASS, C)

    out_e = _lin(jax.nn.gelu(agg_e, approximate=False), p['Wout'][0], p['bout'][0])
    out_p = _lin(jax.nn.gelu(agg_p, approximate=False), p['Wout'][1], p['bout'][1])
    be = jax.nn.sigmoid(p['skip'][0])
    bp = jax.nn.sigmoid(p['skip'][1])
    he = be * out_e + (1.0 - be) * xe
    hp = bp * out_p + (1.0 - bp) * xp
    return he, hp


def _sgformer_trans(x, p):
    x = jax.nn.relu(_ln(_lin(x, p['fc0_W'], p['fc0_b']), p['ln0_g'], p['ln0_b']))
    x0 = x
    q = _lin(x, p['Wq'], p['bq']).reshape(-1, H, C)
    k = _lin(x, p['Wk'], p['bk']).reshape(-1, H, C)
    v = _lin(x, p['Wv'], p['bv']).reshape(-1, H, C)
    q = q / jnp.linalg.norm(q)
    k = k / jnp.linalg.norm(k)
    N = q.shape[0]
    kvs = jnp.einsum('lhm,lhd->hmd', k, v)
    num = jnp.einsum('nhm,hmd->nhd', q, kvs) + N * v
    ks_sum = k.sum(axis=0)
    den = jnp.einsum('nhm,hm->nh', q, ks_sum)[:, :, None] + N
    out = (num / den).mean(axis=1)
    x = 0.5 * out + 0.5 * x0
    x = _ln(x, p['ln1_g'], p['ln1_b'])
    return x


def _mlp(x, W1, b1, W2, b2):
    return _lin(jax.nn.relu(_lin(x, W1, b1)), W2, b2)


def _pallas_identity(x):
    def body(x_ref, o_ref):
        o_ref[...] = x_ref[...]
    return pl.pallas_call(
        body, out_shape=jax.ShapeDtypeStruct(x.shape, x.dtype))(x)


def kernel(x_entity, x_passage, params, ei_ee, ei_ep, ei_pe):
    he, hp = x_entity, x_passage
    for lp in params['hgt']:
        he, hp = _hgt_layer(he, hp, ei_ee, ei_ep, ei_pe, lp)
        he = jax.nn.gelu(he, approximate=False)
        hp = jax.nn.gelu(hp, approximate=False)
    hg = _sgformer_trans(he, params['sg'])
    he = _ln(0.9 * he + 0.1 * hg, params['norm_g'], params['norm_b'])
    he = _pallas_identity(he)
    z_e = _mlp(he, params['pe_W1'], params['pe_b1'], params['pe_W2'], params['pe_b2'])
    z_p = _mlp(hp, params['pp_W1'], params['pp_b1'], params['pp_W2'], params['pp_b2'])
    return z_e, z_p


# speedup vs baseline: 3.2270x; 3.2270x over previous
import functools
import jax
import jax.numpy as jnp
from jax import lax
from jax.experimental import pallas as pl
from jax.experimental.pallas import tpu as pltpu
from jax.experimental.pallas import tpu_sc as plsc

H = 4
C = 128
D = C // H
NE = 40000
NPASS = 10000

NB = 200
CH = 256
CHB = 128
NCORES = 2
NSUB = 16
NW = NCORES * NSUB
ALIGN = NW * CH
ZR = 32
NE_PAD = 40960
NP_PAD = 10240
AW = 48
NEG = -3e38

_mesh = plsc.VectorSubcoreMesh(core_axis_name="c", subcore_axis_name="s")


def _iota16():
    return lax.iota(jnp.int32, 16)


def _full16(v):
    return jnp.full((16,), v, jnp.int32)


def _proj_body(nrel, x_ref, wq_ref, wk_ref, wv_ref, bq_ref, bk_ref, bv_ref,
               a_ref, m_ref, scl_ref, o_ref):
    x = x_ref[...]
    q = jnp.dot(x, wq_ref[...], preferred_element_type=jnp.float32) + bq_ref[...]
    k = jnp.dot(x, wk_ref[...], preferred_element_type=jnp.float32) + bk_ref[...]
    v = jnp.dot(x, wv_ref[...], preferred_element_type=jnp.float32) + bv_ref[...]
    o_ref[0] = q
    for r in range(nrel):
        kcols = []
        vcols = []
        for h in range(H):
            kh = jnp.dot(k[:, h * D:(h + 1) * D], a_ref[r, h],
                         preferred_element_type=jnp.float32)
            kcols.append(kh * scl_ref[0, r * H + h])
            vcols.append(jnp.dot(v[:, h * D:(h + 1) * D], m_ref[r, h],
                                 preferred_element_type=jnp.float32))
        o_ref[1 + 2 * r] = jnp.concatenate(kcols, axis=1)
        o_ref[2 + 2 * r] = jnp.concatenate(vcols, axis=1)


def _proj(x, wq, wk, wv, bq, bk, bv, a_stack, m_stack, scl):
    n = x.shape[0]
    nrel = a_stack.shape[0]
    nproj = 1 + 2 * nrel
    grid = n // NB
    full_w = pl.BlockSpec((C, C), lambda i: (0, 0))
    full_b = pl.BlockSpec((1, C), lambda i: (0, 0))
    return pl.pallas_call(
        functools.partial(_proj_body, nrel),
        grid=(grid,),
        in_specs=[
            pl.BlockSpec((NB, C), lambda i: (i, 0)),
            full_w, full_w, full_w, full_b, full_b, full_b,
            pl.BlockSpec((nrel, H, D, D), lambda i: (0, 0, 0, 0)),
            pl.BlockSpec((nrel, H, D, D), lambda i: (0, 0, 0, 0)),
            pl.BlockSpec(memory_space=pltpu.SMEM),
        ],
        out_specs=pl.BlockSpec((nproj, NB, C), lambda i: (0, i, 0)),
        out_shape=jax.ShapeDtypeStruct((nproj, n, C), jnp.float32),
    )(x, wq, wk, wv, bq.reshape(1, C), bk.reshape(1, C), bv.reshape(1, C),
      a_stack, m_stack, scl)


def _logits_kernel(rels, args, outs, scr):
    cid = lax.axis_index("c")
    sid = lax.axis_index("s")
    wid = sid * NCORES + cid
    srcv, dstv, qg, kg, lb, mxe, mxp, sem = scr
    mxe[...] = jnp.full((16,), NEG, jnp.float32)
    mxp[...] = jnp.full((16,), NEG, jnp.float32)

    for r in rels:
        qtbl = args[r["qtbl"]]
        ktbl = args[r["ktbl"]]
        src = args[r["src"]]
        dst = args[r["dst"]]
        mxref = mxe if r["dst_is_entity"] else mxp
        epw = r["ep"] // NW
        nc = epw // CH
        e_real = r["e_real"]

        def chunk(ci, _):
            base = wid * epw + ci * CH
            pltpu.sync_copy(src.at[pl.ds(base, CH)], srcv)
            pltpu.sync_copy(dst.at[pl.ds(base, CH)], dstv)
            pltpu.async_copy(qtbl.at[0].at[dstv], qg, sem).wait()
            pltpu.async_copy(ktbl.at[r["kj"]].at[srcv], kg, sem).wait()
            for h in range(H):

                def grp(g, _):
                    rows = _iota16() + g * 16
                    acc = jnp.zeros((16,), jnp.float32)
                    for c in range(D):
                        qc = plsc.load_gather(qg, [rows, _full16(h * D + c)])
                        kc = plsc.load_gather(kg, [rows, _full16(h * D + c)])
                        acc = acc + qc * kc
                    gid = base + rows
                    acc = jnp.where(gid < e_real, acc,
                                    jnp.full((16,), NEG, jnp.float32))
                    lb[pl.ds(g * 16, 16)] = acc
                    mxref[...] = jnp.maximum(mxref[...], acc)
                    return 0

                lax.fori_loop(0, CH // 16, grp, 0)
                pltpu.sync_copy(lb, outs[r["lg0"] + h].at[pl.ds(base, CH)])
            return 0

        lax.fori_loop(0, nc, chunk, 0)

    pltpu.sync_copy(mxe, outs[-2].at[pl.ds(wid * 16, 16)])
    pltpu.sync_copy(mxp, outs[-1].at[pl.ds(wid * 16, 16)])


def _run_logits(tbl_e, tbl_p, ee, pe, ep):
    rels = [
        dict(qtbl=0, ktbl=0, kj=1, src=2, dst=3, dst_is_entity=True,
             ep=ee["ep_pad"], e_real=ee["e_real"], lg0=0),
        dict(qtbl=0, ktbl=1, kj=1, src=4, dst=5, dst_is_entity=True,
             ep=pe["ep_pad"], e_real=pe["e_real"], lg0=4),
        dict(qtbl=1, ktbl=0, kj=3, src=6, dst=7, dst_is_entity=False,
             ep=ep["ep_pad"], e_real=ep["e_real"], lg0=8),
    ]
    out_type = ([jax.ShapeDtypeStruct((ee["ep_pad"],), jnp.float32)] * 4
                + [jax.ShapeDtypeStruct((pe["ep_pad"],), jnp.float32)] * 4
                + [jax.ShapeDtypeStruct((ep["ep_pad"],), jnp.float32)] * 4
                + [jax.ShapeDtypeStruct((NW * 16,), jnp.float32)] * 2)

    @functools.partial(
        pl.kernel, mesh=_mesh, out_type=out_type,
        compiler_params=pltpu.CompilerParams(needs_layout_passes=False),
        scratch_types=[
            pltpu.VMEM((CH,), jnp.int32),
            pltpu.VMEM((CH,), jnp.int32),
            pltpu.VMEM((CH, C), jnp.float32),
            pltpu.VMEM((CH, C), jnp.float32),
            pltpu.VMEM((CH,), jnp.float32),
            pltpu.VMEM((16,), jnp.float32),
            pltpu.VMEM((16,), jnp.float32),
            pltpu.SemaphoreType.DMA,
        ],
    )
    def k(te, tp, ees, eed, pes, ped, eps, epd,
          l0, l1, l2, l3, l4, l5, l6, l7, l8, l9, l10, l11, me, mp,
          srcv, dstv, qg, kg, lb, mxe, mxp, sem):
        _logits_kernel(rels, [te, tp, ees, eed, pes, ped, eps, epd],
                       [l0, l1, l2, l3, l4, l5, l6, l7, l8, l9, l10, l11,
                        me, mp],
                       [srcv, dstv, qg, kg, lb, mxe, mxp, sem])

    res = k(tbl_e, tbl_p, ee["src"], ee["dst"], pe["src"], pe["dst"],
            ep["src"], ep["dst"])
    return res[0:4], res[4:8], res[8:12], res[12], res[13]


def _accum_kernel(nseg, nsweeps, rels, args, outs, scr):
    cid = lax.axis_index("c")
    sid = lax.axis_index("s")
    outn, outd = outs
    srcv, dstv, vg, stage, lb0, lb1, lb2, lb3, mxall, zbuf, acc, sem = scr
    lbs = [lb0, lb1, lb2, lb3]
    qrange = nseg // (NCORES * nsweeps)
    rpt = qrange // NSUB

    pltpu.sync_copy(args[0], mxall)
    m = jnp.full((16,), NEG, jnp.float32)
    for i in range(NW):
        m = jnp.maximum(m, mxall[pl.ds(i * 16, 16)])
    mv = jnp.full((16,), jnp.max(m), jnp.float32)

    def zrow(i, _):
        for c in range(C // 16):
            zbuf[i, pl.ds(c * 16, 16)] = jnp.zeros((16,), jnp.float32)
        return 0
    lax.fori_loop(0, ZR, zrow, 0)

    for s in range(nsweeps):
        row0 = (s * NCORES + cid) * qrange
        for mode, out in ((0, outn), (1, outd)):
            for j in range(rpt // ZR):
                pltpu.sync_copy(zbuf, acc.at[pl.ds(sid * rpt + j * ZR, ZR)])
            plsc.subcore_barrier()

            for r in rels:
                vtbl = args[r["vtbl"]]
                src = args[r["src"]]
                dst = args[r["dst"]]
                lgs = [args[r["lg0"] + h] for h in range(H)]
                epw = r["ep"] // NSUB
                nc = epw // CHB

                def chunk(ci, _):
                    base = sid * epw + ci * CHB
                    pltpu.sync_copy(dst.at[pl.ds(base, CHB)], dstv)
                    for h in range(H):
                        pltpu.sync_copy(lgs[h].at[pl.ds(base, CHB)], lbs[h])
                    if mode == 0:
                        pltpu.sync_copy(src.at[pl.ds(base, CHB)], srcv)
                        pltpu.async_copy(vtbl.at[r["vj"]].at[srcv], vg,
                                         sem).wait()

                    def grp(g, _):
                        rows = _iota16() + g * 16
                        dl = dstv[pl.ds(g * 16, 16)] - row0
                        inr = (dl >= 0) & (dl < qrange)
                        dstv[pl.ds(g * 16, 16)] = jnp.where(
                            inr, dl, jnp.zeros((16,), jnp.int32))
                        for h in range(H):
                            w = jnp.exp(lbs[h][pl.ds(g * 16, 16)] - mv)
                            w = jnp.where(inr, w,
                                          jnp.zeros((16,), jnp.float32))
                            for c in range(D):
                                col = _full16(h * D + c)
                                if mode == 0:
                                    vc = plsc.load_gather(vg, [rows, col])
                                    plsc.store_scatter(stage, [rows, col],
                                                       vc * w)
                                else:
                                    plsc.store_scatter(stage, [rows, col], w)
                        return 0

                    lax.fori_loop(0, CHB // 16, grp, 0)
                    pltpu.sync_copy(stage, acc.at[dstv], add=True)
                    return 0

                lax.fori_loop(0, nc, chunk, 0)

            plsc.subcore_barrier()
            off = pl.multiple_of(row0 + sid * rpt, 8)
            pltpu.sync_copy(acc.at[pl.ds(sid * rpt, rpt)],
                            out.at[pl.ds(off, rpt)])
            plsc.subcore_barrier()


def _run_accum(nseg, nsweeps, rels_cfg, arrays):
    qrange = nseg // (NCORES * nsweeps)

    @functools.partial(
        pl.kernel, mesh=_mesh,
        out_type=[jax.ShapeDtypeStruct((nseg, C), jnp.float32),
                  jax.ShapeDtypeStruct((nseg, C), jnp.float32)],
        compiler_params=pltpu.CompilerParams(needs_layout_passes=False),
        scratch_types=[
            pltpu.VMEM((CHB,), jnp.int32),
            pltpu.VMEM((CHB,), jnp.int32),
            pltpu.VMEM((CHB, C), jnp.float32),
            pltpu.VMEM((CHB, C), jnp.float32),
            pltpu.VMEM((CHB,), jnp.float32),
            pltpu.VMEM((CHB,), jnp.float32),
            pltpu.VMEM((CHB,), jnp.float32),
            pltpu.VMEM((CHB,), jnp.float32),
            pltpu.VMEM((NW * 16,), jnp.float32),
            pltpu.VMEM((ZR, C), jnp.float32),
            pltpu.VMEM_SHARED((qrange, C), jnp.float32),
            pltpu.SemaphoreType.DMA,
        ],
    )
    def k(*refs):
        nargs = len(arrays)
        args = refs[:nargs]
        outs = refs[nargs:nargs + 2]
        scr = refs[nargs + 2:]
        _accum_kernel(nseg, nsweeps, rels_cfg, args, outs, scr)

    return k(*arrays)


def _gelu(x):
    return 0.5 * x * (1.0 + lax.erf(x * 0.7071067811865476))


def _finish_body(num_ref, den_ref, x_ref, w_ref, b_ref, beta_ref, o_ref):
    num = num_ref[...]
    den = den_ref[...]
    cols = []
    for h in range(H):
        cols.append(num[:, h * D:(h + 1) * D]
                    / (den[:, h * D:h * D + 1] + 1e-16))
    agg = jnp.concatenate(cols, axis=1)
    out = jnp.dot(_gelu(agg), w_ref[...],
                  preferred_element_type=jnp.float32) + b_ref[...]
    beta = beta_ref[0, 0]
    o_ref[...] = _gelu(beta * out + (1.0 - beta) * x_ref[...])


def _finish(num, den, x, w, b, beta):
    n = x.shape[0]
    grid = n // NB
    return pl.pallas_call(
        _finish_body,
        grid=(grid,),
        in_specs=[
            pl.BlockSpec((NB, C), lambda i: (i, 0)),
            pl.BlockSpec((NB, C), lambda i: (i, 0)),
            pl.BlockSpec((NB, C), lambda i: (i, 0)),
            pl.BlockSpec((C, C), lambda i: (0, 0)),
            pl.BlockSpec((1, C), lambda i: (0, 0)),
            pl.BlockSpec(memory_space=pltpu.SMEM),
        ],
        out_specs=pl.BlockSpec((NB, C), lambda i: (i, 0)),
        out_shape=jax.ShapeDtypeStruct((n, C), jnp.float32),
    )(num, den, x, w, b.reshape(1, C), beta)


def _ln_rows(x, g, b):
    m = jnp.mean(x, axis=1, keepdims=True)
    v = jnp.mean((x - m) ** 2, axis=1, keepdims=True)
    return (x - m) * lax.rsqrt(v + 1e-5) * g + b


def _sg1_body(he_ref, fcw_ref, fcb_ref, lng_ref, lnb_ref,
              wq_ref, bq_ref, wk_ref, bk_ref, wv_ref, bv_ref,
              x0_ref, kvs_ref, ks_ref, ss_ref):
    pi = pl.program_id(0)
    x0 = jnp.maximum(
        _ln_rows(jnp.dot(he_ref[...], fcw_ref[...],
                         preferred_element_type=jnp.float32) + fcb_ref[...],
                 lng_ref[...], lnb_ref[...]), 0.0)
    x0_ref[...] = x0
    qu = jnp.dot(x0, wq_ref[...], preferred_element_type=jnp.float32) + bq_ref[...]
    ku = jnp.dot(x0, wk_ref[...], preferred_element_type=jnp.float32) + bk_ref[...]
    vu = jnp.dot(x0, wv_ref[...], preferred_element_type=jnp.float32) + bv_ref[...]

    @pl.when(pi == 0)
    def _():
        kvs_ref[...] = jnp.zeros_like(kvs_ref)
        ks_ref[...] = jnp.zeros_like(ks_ref)
        ss_ref[...] = jnp.zeros_like(ss_ref)

    ksums = []
    for h in range(H):
        kh = ku[:, h * C:(h + 1) * C]
        vh = vu[:, h * C:(h + 1) * C]
        kvs_ref[h] += lax.dot_general(
            kh, vh, (((0,), (0,)), ((), ())),
            preferred_element_type=jnp.float32)
        ksums.append(jnp.sum(kh, axis=0, keepdims=True))
    ks_ref[...] += jnp.stack(ksums, axis=1)
    ss_ref[...] += jnp.concatenate(
        [jnp.full((1, 64), jnp.sum(qu * qu), jnp.float32),
         jnp.full((1, 64), jnp.sum(ku * ku), jnp.float32)], axis=1)


def _sg2_body(he_ref, x0_ref, kvs_ref, ks_ref, ss_ref,
              wq_ref, bq_ref, wv_ref, bv_ref,
              ln1g_ref, ln1b_ref, ng_ref, nb_ref,
              w1_ref, b1_ref, w2_ref, b2_ref, o_ref):
    x0 = x0_ref[...]
    qu = jnp.dot(x0, wq_ref[...], preferred_element_type=jnp.float32) + bq_ref[...]
    vu = jnp.dot(x0, wv_ref[...], preferred_element_type=jnp.float32) + bv_ref[...]
    ss = ss_ref[...]
    nq = jnp.sqrt(ss[0, 0])
    nk = jnp.sqrt(ss[0, 64])
    ks = ks_ref[...]
    n_nodes = jnp.float32(NE)
    acc = jnp.zeros((NB, C), jnp.float32)
    for h in range(H):
        qh = qu[:, h * C:(h + 1) * C] / nq
        vh = vu[:, h * C:(h + 1) * C]
        num = jnp.dot(qh, kvs_ref[h] / nk,
                      preferred_element_type=jnp.float32) + n_nodes * vh
        ks_row = ks[0, h, :].reshape(1, C) / nk
        den = jnp.sum(qh * ks_row, axis=1, keepdims=True) + n_nodes
        acc += num / den
    out = acc / jnp.float32(H)
    hg = _ln_rows(0.5 * out + 0.5 * x0, ln1g_ref[...], ln1b_ref[...])
    hf = _ln_rows(0.9 * he_ref[...] + 0.1 * hg, ng_ref[...], nb_ref[...])
    z = jnp.maximum(jnp.dot(hf, w1_ref[...],
                            preferred_element_type=jnp.float32) + b1_ref[...],
                    0.0)
    o_ref[...] = jnp.dot(z, w2_ref[...],
                         preferred_element_type=jnp.float32) + b2_ref[...]


def _mlp_body(x_ref, w1_ref, b1_ref, w2_ref, b2_ref, o_ref):
    z = jnp.maximum(jnp.dot(x_ref[...], w1_ref[...],
                            preferred_element_type=jnp.float32) + b1_ref[...],
                    0.0)
    o_ref[...] = jnp.dot(z, w2_ref[...],
                         preferred_element_type=jnp.float32) + b2_ref[...]


def _row_spec():
    return pl.BlockSpec((NB, C), lambda i: (i, 0))


def _w_spec(r, c):
    return pl.BlockSpec((r, c), lambda i: (0, 0))


def _b_spec(c):
    return pl.BlockSpec((1, c), lambda i: (0, 0))


def _sgformer_and_heads(he, sg, norm_g, norm_b, pe_w1, pe_b1, pe_w2, pe_b2):
    grid = NE // NB
    x0, kvs, ks, ss = pl.pallas_call(
        _sg1_body,
        grid=(grid,),
        in_specs=[
            _row_spec(), _w_spec(C, C), _b_spec(C), _b_spec(C), _b_spec(C),
            _w_spec(C, C * H), _b_spec(C * H),
            _w_spec(C, C * H), _b_spec(C * H),
            _w_spec(C, C * H), _b_spec(C * H),
        ],
        out_specs=[
            pl.BlockSpec((NB, C), lambda i: (i, 0)),
            pl.BlockSpec((H, C, C), lambda i: (0, 0, 0)),
            pl.BlockSpec((1, H, C), lambda i: (0, 0, 0)),
            pl.BlockSpec((1, C), lambda i: (0, 0)),
        ],
        out_shape=[
            jax.ShapeDtypeStruct((NE, C), jnp.float32),
            jax.ShapeDtypeStruct((H, C, C), jnp.float32),
            jax.ShapeDtypeStruct((1, H, C), jnp.float32),
            jax.ShapeDtypeStruct((1, C), jnp.float32),
        ],
    )(he, sg['fc0_W'], sg['fc0_b'].reshape(1, C),
      sg['ln0_g'].reshape(1, C), sg['ln0_b'].reshape(1, C),
      sg['Wq'], sg['bq'].reshape(1, C * H),
      sg['Wk'], sg['bk'].reshape(1, C * H),
      sg['Wv'], sg['bv'].reshape(1, C * H))

    z_e = pl.pallas_call(
        _sg2_body,
        grid=(grid,),
        in_specs=[
            _row_spec(), _row_spec(),
            pl.BlockSpec((H, C, C), lambda i: (0, 0, 0)),
            pl.BlockSpec((1, H, C), lambda i: (0, 0, 0)),
            _b_spec(C),
            _w_spec(C, C * H), _b_spec(C * H),
            _w_spec(C, C * H), _b_spec(C * H),
            _b_spec(C), _b_spec(C), _b_spec(C), _b_spec(C),
            _w_spec(C, C), _b_spec(C), _w_spec(C, C), _b_spec(C),
        ],
        out_specs=pl.BlockSpec((NB, C), lambda i: (i, 0)),
        out_shape=jax.ShapeDtypeStruct((NE, C), jnp.float32),
    )(he, x0, kvs, ks, ss,
      sg['Wq'], sg['bq'].reshape(1, C * H),
      sg['Wv'], sg['bv'].reshape(1, C * H),
      sg['ln1_g'].reshape(1, C), sg['ln1_b'].reshape(1, C),
      norm_g.reshape(1, C), norm_b.reshape(1, C),
      pe_w1, pe_b1.reshape(1, C), pe_w2, pe_b2.reshape(1, C))
    return z_e


def _mlp_call(x, w1, b1, w2, b2):
    n = x.shape[0]
    return pl.pallas_call(
        _mlp_body,
        grid=(n // NB,),
        in_specs=[_row_spec(), _w_spec(C, C), _b_spec(C),
                  _w_spec(C, C), _b_spec(C)],
        out_specs=pl.BlockSpec((NB, C), lambda i: (i, 0)),
        out_shape=jax.ShapeDtypeStruct((n, C), jnp.float32),
    )(x, w1, b1.reshape(1, C), w2, b2.reshape(1, C))


def _pad_edges(ei):
    e = ei.shape[1]
    ep = ((e + ALIGN - 1) // ALIGN) * ALIGN
    pad = ep - e
    src = jnp.concatenate(
        [ei[0].astype(jnp.int32), jnp.zeros((pad,), jnp.int32)])
    dst = jnp.concatenate(
        [ei[1].astype(jnp.int32), jnp.zeros((pad,), jnp.int32)])
    return dict(src=src, dst=dst, ep_pad=ep, e_real=e)


def _hgt_layer(xe, xp, ee, pe, ep, p):
    sd = float(D) ** 0.5
    scl_e = jnp.concatenate([p['p_rel'][0] / sd, p['p_rel'][1] / sd]).reshape(1, 2 * H)
    scl_p = (p['p_rel'][2] / sd).reshape(1, H)
    tbl_e = _proj(xe, p['Wq'][0], p['Wk'][0], p['Wv'][0],
                  p['bq'][0], p['bk'][0], p['bv'][0],
                  jnp.stack([p['a_rel'][0], p['a_rel'][1]]),
                  jnp.stack([p['m_rel'][0], p['m_rel'][1]]), scl_e)
    tbl_p = _proj(xp, p['Wq'][1], p['Wk'][1], p['Wv'][1],
                  p['bq'][1], p['bk'][1], p['bv'][1],
                  p['a_rel'][2][None], p['m_rel'][2][None], scl_p)

    lg_ee, lg_pe, lg_ep, mx_e, mx_p = _run_logits(tbl_e, tbl_p, ee, pe, ep)

    rels_e = [
        dict(vtbl=1, vj=2, src=3, dst=4, lg0=5, ep=ee["ep_pad"]),
        dict(vtbl=2, vj=2, src=9, dst=10, lg0=11, ep=pe["ep_pad"]),
    ]
    args_e = ([mx_e, tbl_e, tbl_p, ee["src"], ee["dst"]] + list(lg_ee)
              + [pe["src"], pe["dst"]] + list(lg_pe))
    num_e, den_e = _run_accum(NE_PAD, 2, rels_e, args_e)

    rels_p = [dict(vtbl=1, vj=4, src=2, dst=3, lg0=4, ep=ep["ep_pad"])]
    args_p = [mx_p, tbl_e, ep["src"], ep["dst"]] + list(lg_ep)
    num_p, den_p = _run_accum(NP_PAD, 1, rels_p, args_p)

    be = jax.nn.sigmoid(p['skip'][0]).reshape(1, 1)
    bp = jax.nn.sigmoid(p['skip'][1]).reshape(1, 1)
    he = _finish(num_e, den_e, xe, p['Wout'][0], p['bout'][0], be)
    hp = _finish(num_p, den_p, xp, p['Wout'][1], p['bout'][1], bp)
    return he, hp


def kernel(x_entity, x_passage, params, ei_ee, ei_ep, ei_pe):
    ee = _pad_edges(ei_ee)
    ep = _pad_edges(ei_ep)
    pe = _pad_edges(ei_pe)
    he, hp = x_entity, x_passage
    for lp in params['hgt']:
        he, hp = _hgt_layer(he, hp, ee, pe, ep, lp)
    z_e = _sgformer_and_heads(he, params['sg'], params['norm_g'],
                              params['norm_b'], params['pe_W1'],
                              params['pe_b1'], params['pe_W2'],
                              params['pe_b2'])
    z_p = _mlp_call(hp, params['pp_W1'], params['pp_b1'],
                    params['pp_W2'], params['pp_b2'])
    return z_e, z_p


# speedup vs baseline: 3.9759x; 1.2321x over previous
import functools
import jax
import jax.numpy as jnp
from jax import lax
from jax.experimental import pallas as pl
from jax.experimental.pallas import tpu as pltpu
from jax.experimental.pallas import tpu_sc as plsc

H = 4
C = 128
D = C // H
NE = 40000
NPASS = 10000

NB = 200
CH = 256
CHB = 128
NCORES = 2
NSUB = 16
NW = NCORES * NSUB
ALIGN = NW * CH
ZR = 32
NE_PAD = 40960
NP_PAD = 10240
AW = 48
NEG = -3e38

def _sc_mesh():
    return plsc.VectorSubcoreMesh(core_axis_name="c", subcore_axis_name="s")


def _iota16():
    return lax.iota(jnp.int32, 16)


def _full16(v):
    return jnp.full((16,), v, jnp.int32)


def _proj_body(nrel, x_ref, wq_ref, wk_ref, wv_ref, bq_ref, bk_ref, bv_ref,
               a_ref, m_ref, scl_ref, o_ref):
    x = x_ref[...]
    q = jnp.dot(x, wq_ref[...], preferred_element_type=jnp.float32) + bq_ref[...]
    k = jnp.dot(x, wk_ref[...], preferred_element_type=jnp.float32) + bk_ref[...]
    v = jnp.dot(x, wv_ref[...], preferred_element_type=jnp.float32) + bv_ref[...]
    o_ref[0] = q
    for r in range(nrel):
        kcols = []
        vcols = []
        for h in range(H):
            kh = jnp.dot(k[:, h * D:(h + 1) * D], a_ref[r, h],
                         preferred_element_type=jnp.float32)
            kcols.append(kh * scl_ref[0, r * H + h])
            vcols.append(jnp.dot(v[:, h * D:(h + 1) * D], m_ref[r, h],
                                 preferred_element_type=jnp.float32))
        o_ref[1 + 2 * r] = jnp.concatenate(kcols, axis=1)
        o_ref[2 + 2 * r] = jnp.concatenate(vcols, axis=1)


def _proj(x, wq, wk, wv, bq, bk, bv, a_stack, m_stack, scl):
    n = x.shape[0]
    nrel = a_stack.shape[0]
    nproj = 1 + 2 * nrel
    grid = n // NB
    full_w = pl.BlockSpec((C, C), lambda i: (0, 0))
    full_b = pl.BlockSpec((1, C), lambda i: (0, 0))
    return pl.pallas_call(
        functools.partial(_proj_body, nrel),
        grid=(grid,),
        in_specs=[
            pl.BlockSpec((NB, C), lambda i: (i, 0)),
            full_w, full_w, full_w, full_b, full_b, full_b,
            pl.BlockSpec((nrel, H, D, D), lambda i: (0, 0, 0, 0)),
            pl.BlockSpec((nrel, H, D, D), lambda i: (0, 0, 0, 0)),
            pl.BlockSpec(memory_space=pltpu.SMEM),
        ],
        out_specs=pl.BlockSpec((nproj, NB, C), lambda i: (0, i, 0)),
        out_shape=jax.ShapeDtypeStruct((nproj, n, C), jnp.float32),
    )(x, wq, wk, wv, bq.reshape(1, C), bk.reshape(1, C), bv.reshape(1, C),
      a_stack, m_stack, scl)


def _logits_kernel(rels, args, outs, scr):
    cid = lax.axis_index("c")
    sid = lax.axis_index("s")
    wid = sid * NCORES + cid
    srcv, dstv, qg, kg, lb, mxe, mxp, sem = scr
    mxe[...] = jnp.full((16,), NEG, jnp.float32)
    mxp[...] = jnp.full((16,), NEG, jnp.float32)

    for r in rels:
        qtbl = args[r["qtbl"]]
        ktbl = args[r["ktbl"]]
        src = args[r["src"]]
        dst = args[r["dst"]]
        mxref = mxe if r["dst_is_entity"] else mxp
        epw = r["ep"] // NW
        nc = epw // CH
        e_real = r["e_real"]

        def chunk(ci, _):
            base = wid * epw + ci * CH
            pltpu.sync_copy(src.at[pl.ds(base, CH)], srcv)
            pltpu.sync_copy(dst.at[pl.ds(base, CH)], dstv)
            pltpu.async_copy(qtbl.at[0].at[dstv], qg, sem).wait()
            pltpu.async_copy(ktbl.at[r["kj"]].at[srcv], kg, sem).wait()
            for h in range(H):

                def grp(g, _):
                    rows = _iota16() + g * 16
                    acc = jnp.zeros((16,), jnp.float32)
                    for c in range(D):
                        qc = plsc.load_gather(qg, [rows, _full16(h * D + c)])
                        kc = plsc.load_gather(kg, [rows, _full16(h * D + c)])
                        acc = acc + qc * kc
                    gid = base + rows
                    acc = jnp.where(gid < e_real, acc,
                                    jnp.full((16,), NEG, jnp.float32))
                    lb[pl.ds(g * 16, 16)] = acc
                    mxref[...] = jnp.maximum(mxref[...], acc)
                    return 0

                lax.fori_loop(0, CH // 16, grp, 0)
                pltpu.sync_copy(lb, outs[r["lg0"] + h].at[pl.ds(base, CH)])
            return 0

        lax.fori_loop(0, nc, chunk, 0)

    pltpu.sync_copy(mxe, outs[-2].at[pl.ds(wid * 16, 16)])
    pltpu.sync_copy(mxp, outs[-1].at[pl.ds(wid * 16, 16)])


def _run_logits(tbl_e, tbl_p, ee, pe, ep):
    rels = [
        dict(qtbl=0, ktbl=0, kj=1, src=2, dst=3, dst_is_entity=True,
             ep=ee["ep_pad"], e_real=ee["e_real"], lg0=0),
        dict(qtbl=0, ktbl=1, kj=1, src=4, dst=5, dst_is_entity=True,
             ep=pe["ep_pad"], e_real=pe["e_real"], lg0=4),
        dict(qtbl=1, ktbl=0, kj=3, src=6, dst=7, dst_is_entity=False,
             ep=ep["ep_pad"], e_real=ep["e_real"], lg0=8),
    ]
    out_type = ([jax.ShapeDtypeStruct((ee["ep_pad"],), jnp.float32)] * 4
                + [jax.ShapeDtypeStruct((pe["ep_pad"],), jnp.float32)] * 4
                + [jax.ShapeDtypeStruct((ep["ep_pad"],), jnp.float32)] * 4
                + [jax.ShapeDtypeStruct((NW * 16,), jnp.float32)] * 2)

    @functools.partial(
        pl.kernel, mesh=_sc_mesh(), out_type=out_type,
        compiler_params=pltpu.CompilerParams(needs_layout_passes=False),
        scratch_types=[
            pltpu.VMEM((CH,), jnp.int32),
            pltpu.VMEM((CH,), jnp.int32),
            pltpu.VMEM((CH, C), jnp.float32),
            pltpu.VMEM((CH, C), jnp.float32),
            pltpu.VMEM((CH,), jnp.float32),
            pltpu.VMEM((16,), jnp.float32),
            pltpu.VMEM((16,), jnp.float32),
            pltpu.SemaphoreType.DMA,
        ],
    )
    def k(te, tp, ees, eed, pes, ped, eps, epd,
          l0, l1, l2, l3, l4, l5, l6, l7, l8, l9, l10, l11, me, mp,
          srcv, dstv, qg, kg, lb, mxe, mxp, sem):
        _logits_kernel(rels, [te, tp, ees, eed, pes, ped, eps, epd],
                       [l0, l1, l2, l3, l4, l5, l6, l7, l8, l9, l10, l11,
                        me, mp],
                       [srcv, dstv, qg, kg, lb, mxe, mxp, sem])

    res = k(tbl_e, tbl_p, ee["src"], ee["dst"], pe["src"], pe["dst"],
            ep["src"], ep["dst"])
    return res[0:4], res[4:8], res[8:12], res[12], res[13]


def _accum_kernel(nseg, nsweeps, packed_den, rels, args, outs, scr):
    cid = lax.axis_index("c")
    sid = lax.axis_index("s")
    outn, outd = outs
    srcv, dstv, vg, stage, lb0, lb1, lb2, lb3, mxall, zbuf, acc, sem = scr
    lbs = [lb0, lb1, lb2, lb3]
    qrange = nseg // (NCORES * nsweeps)
    rpt = qrange // NSUB

    pltpu.sync_copy(args[0], mxall)
    m = jnp.full((16,), NEG, jnp.float32)
    for i in range(NW):
        m = jnp.maximum(m, mxall[pl.ds(i * 16, 16)])
    mv = jnp.full((16,), jnp.max(m), jnp.float32)

    def zrow(i, _):
        for c in range(C // 16):
            zbuf[i, pl.ds(c * 16, 16)] = jnp.zeros((16,), jnp.float32)
        return 0
    lax.fori_loop(0, ZR, zrow, 0)

    if packed_den:
        wid32 = sid * NCORES + cid
        rptd = (nseg // 4) // NSUB
        for j in range(rptd // ZR):
            pltpu.sync_copy(zbuf, acc.at[pl.ds(sid * rptd + j * ZR, ZR)])
        plsc.subcore_barrier()

        def sclr(i, _):
            for cb in range(C // 16):
                stage[i, pl.ds(cb * 16, 16)] = jnp.zeros((16,), jnp.float32)
            return 0
        lax.fori_loop(0, CHB, sclr, 0)

        for r in rels:
            dst = args[r["dst"]]
            lgs = [args[r["lg0"] + h] for h in range(H)]
            epw = r["ep"] // NW
            nc = epw // CHB

            def chunk_d(ci, _):
                base = wid32 * epw + ci * CHB
                pltpu.sync_copy(dst.at[pl.ds(base, CHB)], dstv)
                for h in range(H):
                    pltpu.sync_copy(lgs[h].at[pl.ds(base, CHB)], lbs[h])

                def grp(g, _):
                    rows = _iota16() + g * 16
                    dl = dstv[pl.ds(g * 16, 16)]
                    dstv[pl.ds(g * 16, 16)] = lax.shift_right_logical(
                        dl, jnp.full((16,), 2, jnp.int32))
                    sub = (dl & jnp.full((16,), 3, jnp.int32)) * 4
                    for k in range(16):
                        plsc.store_scatter(stage, [rows, _full16(k)],
                                           jnp.zeros((16,), jnp.float32))
                    for h in range(H):
                        w = jnp.exp(lbs[h][pl.ds(g * 16, 16)] - mv)
                        plsc.store_scatter(stage, [rows, sub + h], w)
                    return 0

                lax.fori_loop(0, CHB // 16, grp, 0)
                pltpu.sync_copy(stage, acc.at[dstv], add=True)
                return 0

            lax.fori_loop(0, nc, chunk_d, 0)
        plsc.subcore_barrier()
        offd = pl.multiple_of(sid * rptd, 8)
        pltpu.sync_copy(acc.at[pl.ds(sid * rptd, rptd)],
                        outd.at[cid, pl.ds(offd, rptd)])
        plsc.subcore_barrier()
        mode_list = ((0, outn),)
    else:
        mode_list = ((0, outn), (1, outd))

    for s in range(nsweeps):
        row0 = (s * NCORES + cid) * qrange
        for mode, out in mode_list:
            for j in range(rpt // ZR):
                pltpu.sync_copy(zbuf, acc.at[pl.ds(sid * rpt + j * ZR, ZR)])
            plsc.subcore_barrier()

            for r in rels:
                vtbl = args[r["vtbl"]]
                src = args[r["src"]]
                dst = args[r["dst"]]
                lgs = [args[r["lg0"] + h] for h in range(H)]
                epw = r["ep"] // NSUB
                nc = epw // CHB

                def chunk(ci, _):
                    base = sid * epw + ci * CHB
                    pltpu.sync_copy(dst.at[pl.ds(base, CHB)], dstv)
                    for h in range(H):
                        pltpu.sync_copy(lgs[h].at[pl.ds(base, CHB)], lbs[h])
                    if mode == 0:
                        pltpu.sync_copy(src.at[pl.ds(base, CHB)], srcv)
                        pltpu.async_copy(vtbl.at[r["vj"]].at[srcv], vg,
                                         sem).wait()

                    def grp(g, _):
                        rows = _iota16() + g * 16
                        dl = dstv[pl.ds(g * 16, 16)] - row0
                        inr = (dl >= 0) & (dl < qrange)
                        dstv[pl.ds(g * 16, 16)] = jnp.where(
                            inr, dl, jnp.zeros((16,), jnp.int32))
                        for h in range(H):
                            w = jnp.exp(lbs[h][pl.ds(g * 16, 16)] - mv)
                            w = jnp.where(inr, w,
                                          jnp.zeros((16,), jnp.float32))
                            for c in range(D):
                                col = _full16(h * D + c)
                                if mode == 0:
                                    vc = plsc.load_gather(vg, [rows, col])
                                    plsc.store_scatter(stage, [rows, col],
                                                       vc * w)
                                else:
                                    plsc.store_scatter(stage, [rows, col], w)
                        return 0

                    lax.fori_loop(0, CHB // 16, grp, 0)
                    pltpu.sync_copy(stage, acc.at[dstv], add=True)
                    return 0

                lax.fori_loop(0, nc, chunk, 0)

            plsc.subcore_barrier()
            off = pl.multiple_of(row0 + sid * rpt, 8)
            pltpu.sync_copy(acc.at[pl.ds(sid * rpt, rpt)],
                            out.at[pl.ds(off, rpt)])
            plsc.subcore_barrier()


def _run_accum(nseg, nsweeps, packed_den, rels_cfg, arrays):
    qrange = nseg // (NCORES * nsweeps)
    if packed_den:
        outd_t = jax.ShapeDtypeStruct((NCORES, nseg // 4, C), jnp.float32)
    else:
        outd_t = jax.ShapeDtypeStruct((nseg, C), jnp.float32)

    @functools.partial(
        pl.kernel, mesh=_sc_mesh(),
        out_type=[jax.ShapeDtypeStruct((nseg, C), jnp.float32), outd_t],
        compiler_params=pltpu.CompilerParams(needs_layout_passes=False),
        scratch_types=[
            pltpu.VMEM((CHB,), jnp.int32),
            pltpu.VMEM((CHB,), jnp.int32),
            pltpu.VMEM((CHB, C), jnp.float32),
            pltpu.VMEM((CHB, C), jnp.float32),
            pltpu.VMEM((CHB,), jnp.float32),
            pltpu.VMEM((CHB,), jnp.float32),
            pltpu.VMEM((CHB,), jnp.float32),
            pltpu.VMEM((CHB,), jnp.float32),
            pltpu.VMEM((NW * 16,), jnp.float32),
            pltpu.VMEM((ZR, C), jnp.float32),
            pltpu.VMEM_SHARED((qrange, C), jnp.float32),
            pltpu.SemaphoreType.DMA,
        ],
    )
    def k(*refs):
        nargs = len(arrays)
        args = refs[:nargs]
        outs = refs[nargs:nargs + 2]
        scr = refs[nargs + 2:]
        _accum_kernel(nseg, nsweeps, packed_den, rels_cfg, args, outs, scr)

    return k(*arrays)


def _gelu(x):
    return 0.5 * x * (1.0 + lax.erf(x * 0.7071067811865476))


def _finish_body(num_ref, den_ref, x_ref, w_ref, b_ref, beta_ref, o_ref):
    num = num_ref[...]
    den = den_ref[...]
    cols = []
    for h in range(H):
        cols.append(num[:, h * D:(h + 1) * D]
                    / (den[:, h * D:h * D + 1] + 1e-16))
    agg = jnp.concatenate(cols, axis=1)
    out = jnp.dot(_gelu(agg), w_ref[...],
                  preferred_element_type=jnp.float32) + b_ref[...]
    beta = beta_ref[0, 0]
    o_ref[...] = _gelu(beta * out + (1.0 - beta) * x_ref[...])


NBE = 160


def _finish_e_body(num_ref, den_ref, x_ref, w_ref, b_ref, beta_ref, o_ref):
    num = num_ref[...]
    dsum = den_ref[0] + den_ref[1]
    den_rep = jnp.broadcast_to(dsum[:, None, :],
                               (NBE // 4, 4, C)).reshape(NBE, C)
    rowmod = lax.broadcasted_iota(jnp.int32, (NBE, 1), 0) % 4
    cols = []
    for h in range(H):
        den_h = jnp.zeros((NBE, 1), jnp.float32)
        for j in range(4):
            den_h += (den_rep[:, j * 4 + h:j * 4 + h + 1]
                      * (rowmod == j).astype(jnp.float32))
        cols.append(num[:, h * D:(h + 1) * D] / (den_h + 1e-16))
    agg = jnp.concatenate(cols, axis=1)
    out = jnp.dot(_gelu(agg), w_ref[...],
                  preferred_element_type=jnp.float32) + b_ref[...]
    beta = beta_ref[0, 0]
    o_ref[...] = _gelu(beta * out + (1.0 - beta) * x_ref[...])


def _finish_e(num, den, x, w, b, beta):
    n = x.shape[0]
    grid = n // NBE
    return pl.pallas_call(
        _finish_e_body,
        grid=(grid,),
        in_specs=[
            pl.BlockSpec((NBE, C), lambda i: (i, 0)),
            pl.BlockSpec((NCORES, NBE // 4, C), lambda i: (0, i, 0)),
            pl.BlockSpec((NBE, C), lambda i: (i, 0)),
            pl.BlockSpec((C, C), lambda i: (0, 0)),
            pl.BlockSpec((1, C), lambda i: (0, 0)),
            pl.BlockSpec(memory_space=pltpu.SMEM),
        ],
        out_specs=pl.BlockSpec((NBE, C), lambda i: (i, 0)),
        out_shape=jax.ShapeDtypeStruct((n, C), jnp.float32),
    )(num, den, x, w, b.reshape(1, C), beta)


def _finish(num, den, x, w, b, beta):
    n = x.shape[0]
    grid = n // NB
    return pl.pallas_call(
        _finish_body,
        grid=(grid,),
        in_specs=[
            pl.BlockSpec((NB, C), lambda i: (i, 0)),
            pl.BlockSpec((NB, C), lambda i: (i, 0)),
            pl.BlockSpec((NB, C), lambda i: (i, 0)),
            pl.BlockSpec((C, C), lambda i: (0, 0)),
            pl.BlockSpec((1, C), lambda i: (0, 0)),
            pl.BlockSpec(memory_space=pltpu.SMEM),
        ],
        out_specs=pl.BlockSpec((NB, C), lambda i: (i, 0)),
        out_shape=jax.ShapeDtypeStruct((n, C), jnp.float32),
    )(num, den, x, w, b.reshape(1, C), beta)


def _ln_rows(x, g, b):
    m = jnp.mean(x, axis=1, keepdims=True)
    v = jnp.mean((x - m) ** 2, axis=1, keepdims=True)
    return (x - m) * lax.rsqrt(v + 1e-5) * g + b


def _sg1_body(he_ref, fcw_ref, fcb_ref, lng_ref, lnb_ref,
              wq_ref, bq_ref, wk_ref, bk_ref, wv_ref, bv_ref,
              x0_ref, kvs_ref, ks_ref, ss_ref):
    pi = pl.program_id(0)
    x0 = jnp.maximum(
        _ln_rows(jnp.dot(he_ref[...], fcw_ref[...],
                         preferred_element_type=jnp.float32) + fcb_ref[...],
                 lng_ref[...], lnb_ref[...]), 0.0)
    x0_ref[...] = x0
    qu = jnp.dot(x0, wq_ref[...], preferred_element_type=jnp.float32) + bq_ref[...]
    ku = jnp.dot(x0, wk_ref[...], preferred_element_type=jnp.float32) + bk_ref[...]
    vu = jnp.dot(x0, wv_ref[...], preferred_element_type=jnp.float32) + bv_ref[...]

    @pl.when(pi == 0)
    def _():
        kvs_ref[...] = jnp.zeros_like(kvs_ref)
        ks_ref[...] = jnp.zeros_like(ks_ref)
        ss_ref[...] = jnp.zeros_like(ss_ref)

    ksums = []
    for h in range(H):
        kh = ku[:, h * C:(h + 1) * C]
        vh = vu[:, h * C:(h + 1) * C]
        kvs_ref[h] += lax.dot_general(
            kh, vh, (((0,), (0,)), ((), ())),
            preferred_element_type=jnp.float32)
        ksums.append(jnp.sum(kh, axis=0, keepdims=True))
    ks_ref[...] += jnp.stack(ksums, axis=1)
    ss_ref[...] += jnp.concatenate(
        [jnp.full((1, 64), jnp.sum(qu * qu), jnp.float32),
         jnp.full((1, 64), jnp.sum(ku * ku), jnp.float32)], axis=1)


def _sg2_body(he_ref, x0_ref, kvs_ref, ks_ref, ss_ref,
              wq_ref, bq_ref, wv_ref, bv_ref,
              ln1g_ref, ln1b_ref, ng_ref, nb_ref,
              w1_ref, b1_ref, w2_ref, b2_ref, o_ref):
    x0 = x0_ref[...]
    qu = jnp.dot(x0, wq_ref[...], preferred_element_type=jnp.float32) + bq_ref[...]
    vu = jnp.dot(x0, wv_ref[...], preferred_element_type=jnp.float32) + bv_ref[...]
    ss = ss_ref[...]
    nq = jnp.sqrt(ss[0, 0])
    nk = jnp.sqrt(ss[0, 64])
    ks = ks_ref[...]
    n_nodes = jnp.float32(NE)
    acc = jnp.zeros((NB, C), jnp.float32)
    for h in range(H):
        qh = qu[:, h * C:(h + 1) * C] / nq
        vh = vu[:, h * C:(h + 1) * C]
        num = jnp.dot(qh, kvs_ref[h] / nk,
                      preferred_element_type=jnp.float32) + n_nodes * vh
        ks_row = ks[0, h, :].reshape(1, C) / nk
        den = jnp.sum(qh * ks_row, axis=1, keepdims=True) + n_nodes
        acc += num / den
    out = acc / jnp.float32(H)
    hg = _ln_rows(0.5 * out + 0.5 * x0, ln1g_ref[...], ln1b_ref[...])
    hf = _ln_rows(0.9 * he_ref[...] + 0.1 * hg, ng_ref[...], nb_ref[...])
    z = jnp.maximum(jnp.dot(hf, w1_ref[...],
                            preferred_element_type=jnp.float32) + b1_ref[...],
                    0.0)
    o_ref[...] = jnp.dot(z, w2_ref[...],
                         preferred_element_type=jnp.float32) + b2_ref[...]


def _mlp_body(x_ref, w1_ref, b1_ref, w2_ref, b2_ref, o_ref):
    z = jnp.maximum(jnp.dot(x_ref[...], w1_ref[...],
                            preferred_element_type=jnp.float32) + b1_ref[...],
                    0.0)
    o_ref[...] = jnp.dot(z, w2_ref[...],
                         preferred_element_type=jnp.float32) + b2_ref[...]


def _row_spec():
    return pl.BlockSpec((NB, C), lambda i: (i, 0))


def _w_spec(r, c):
    return pl.BlockSpec((r, c), lambda i: (0, 0))


def _b_spec(c):
    return pl.BlockSpec((1, c), lambda i: (0, 0))


def _sgformer_and_heads(he, sg, norm_g, norm_b, pe_w1, pe_b1, pe_w2, pe_b2):
    grid = NE // NB
    x0, kvs, ks, ss = pl.pallas_call(
        _sg1_body,
        grid=(grid,),
        in_specs=[
            _row_spec(), _w_spec(C, C), _b_spec(C), _b_spec(C), _b_spec(C),
            _w_spec(C, C * H), _b_spec(C * H),
            _w_spec(C, C * H), _b_spec(C * H),
            _w_spec(C, C * H), _b_spec(C * H),
        ],
        out_specs=[
            pl.BlockSpec((NB, C), lambda i: (i, 0)),
            pl.BlockSpec((H, C, C), lambda i: (0, 0, 0)),
            pl.BlockSpec((1, H, C), lambda i: (0, 0, 0)),
            pl.BlockSpec((1, C), lambda i: (0, 0)),
        ],
        out_shape=[
            jax.ShapeDtypeStruct((NE, C), jnp.float32),
            jax.ShapeDtypeStruct((H, C, C), jnp.float32),
            jax.ShapeDtypeStruct((1, H, C), jnp.float32),
            jax.ShapeDtypeStruct((1, C), jnp.float32),
        ],
    )(he, sg['fc0_W'], sg['fc0_b'].reshape(1, C),
      sg['ln0_g'].reshape(1, C), sg['ln0_b'].reshape(1, C),
      sg['Wq'], sg['bq'].reshape(1, C * H),
      sg['Wk'], sg['bk'].reshape(1, C * H),
      sg['Wv'], sg['bv'].reshape(1, C * H))

    z_e = pl.pallas_call(
        _sg2_body,
        grid=(grid,),
        in_specs=[
            _row_spec(), _row_spec(),
            pl.BlockSpec((H, C, C), lambda i: (0, 0, 0)),
            pl.BlockSpec((1, H, C), lambda i: (0, 0, 0)),
            _b_spec(C),
            _w_spec(C, C * H), _b_spec(C * H),
            _w_spec(C, C * H), _b_spec(C * H),
            _b_spec(C), _b_spec(C), _b_spec(C), _b_spec(C),
            _w_spec(C, C), _b_spec(C), _w_spec(C, C), _b_spec(C),
        ],
        out_specs=pl.BlockSpec((NB, C), lambda i: (i, 0)),
        out_shape=jax.ShapeDtypeStruct((NE, C), jnp.float32),
    )(he, x0, kvs, ks, ss,
      sg['Wq'], sg['bq'].reshape(1, C * H),
      sg['Wv'], sg['bv'].reshape(1, C * H),
      sg['ln1_g'].reshape(1, C), sg['ln1_b'].reshape(1, C),
      norm_g.reshape(1, C), norm_b.reshape(1, C),
      pe_w1, pe_b1.reshape(1, C), pe_w2, pe_b2.reshape(1, C))
    return z_e


def _mlp_call(x, w1, b1, w2, b2):
    n = x.shape[0]
    return pl.pallas_call(
        _mlp_body,
        grid=(n // NB,),
        in_specs=[_row_spec(), _w_spec(C, C), _b_spec(C),
                  _w_spec(C, C), _b_spec(C)],
        out_specs=pl.BlockSpec((NB, C), lambda i: (i, 0)),
        out_shape=jax.ShapeDtypeStruct((n, C), jnp.float32),
    )(x, w1, b1.reshape(1, C), w2, b2.reshape(1, C))


def _pad_edges(ei):
    e = ei.shape[1]
    ep = ((e + ALIGN - 1) // ALIGN) * ALIGN
    pad = ep - e
    src = jnp.concatenate(
        [ei[0].astype(jnp.int32), jnp.zeros((pad,), jnp.int32)])
    dst = jnp.concatenate(
        [ei[1].astype(jnp.int32), jnp.zeros((pad,), jnp.int32)])
    return dict(src=src, dst=dst, ep_pad=ep, e_real=e)


def _hgt_layer(xe, xp, ee, pe, ep, p):
    sd = float(D) ** 0.5
    scl_e = jnp.concatenate([p['p_rel'][0] / sd, p['p_rel'][1] / sd]).reshape(1, 2 * H)
    scl_p = (p['p_rel'][2] / sd).reshape(1, H)
    tbl_e = _proj(xe, p['Wq'][0], p['Wk'][0], p['Wv'][0],
                  p['bq'][0], p['bk'][0], p['bv'][0],
                  jnp.stack([p['a_rel'][0], p['a_rel'][1]]),
                  jnp.stack([p['m_rel'][0], p['m_rel'][1]]), scl_e)
    tbl_p = _proj(xp, p['Wq'][1], p['Wk'][1], p['Wv'][1],
                  p['bq'][1], p['bk'][1], p['bv'][1],
                  p['a_rel'][2][None], p['m_rel'][2][None], scl_p)

    lg_ee, lg_pe, lg_ep, mx_e, mx_p = _run_logits(tbl_e, tbl_p, ee, pe, ep)

    rels_e = [
        dict(vtbl=1, vj=2, src=3, dst=4, lg0=5, ep=ee["ep_pad"]),
        dict(vtbl=2, vj=2, src=9, dst=10, lg0=11, ep=pe["ep_pad"]),
    ]
    args_e = ([mx_e, tbl_e, tbl_p, ee["src"], ee["dst"]] + list(lg_ee)
              + [pe["src"], pe["dst"]] + list(lg_pe))
    num_e, den_e = _run_accum(NE_PAD, 2, True, rels_e, args_e)

    rels_p = [dict(vtbl=1, vj=4, src=2, dst=3, lg0=4, ep=ep["ep_pad"])]
    args_p = [mx_p, tbl_e, ep["src"], ep["dst"]] + list(lg_ep)
    num_p, den_p = _run_accum(NP_PAD, 1, False, rels_p, args_p)

    be = jax.nn.sigmoid(p['skip'][0]).reshape(1, 1)
    bp = jax.nn.sigmoid(p['skip'][1]).reshape(1, 1)
    he = _finish_e(num_e, den_e, xe, p['Wout'][0], p['bout'][0], be)
    hp = _finish(num_p, den_p, xp, p['Wout'][1], p['bout'][1], bp)
    return he, hp


def kernel(x_entity, x_passage, params, ei_ee, ei_ep, ei_pe):
    ee = _pad_edges(ei_ee)
    ep = _pad_edges(ei_ep)
    pe = _pad_edges(ei_pe)
    he, hp = x_entity, x_passage
    for lp in params['hgt']:
        he, hp = _hgt_layer(he, hp, ee, pe, ep, lp)
    z_e = _sgformer_and_heads(he, params['sg'], params['norm_g'],
                              params['norm_b'], params['pe_W1'],
                              params['pe_b1'], params['pe_W2'],
                              params['pe_b2'])
    z_p = _mlp_call(hp, params['pp_W1'], params['pp_b1'],
                    params['pp_W2'], params['pp_b2'])
    return z_e, z_p


# speedup vs baseline: 4.3528x; 1.0948x over previous
import functools
import jax
import jax.numpy as jnp
from jax import lax
from jax.experimental import pallas as pl
from jax.experimental.pallas import tpu as pltpu
from jax.experimental.pallas import tpu_sc as plsc

H = 4
C = 128
D = C // H
NE = 40000
NPASS = 10000

NB = 200
CH = 256
CHB = 128
NCORES = 2
NSUB = 16
NW = NCORES * NSUB
ALIGN = NW * CH
ZR = 32
NE_PAD = 40960
NP_PAD = 10240
AW = 48
NEG = -3e38

def _sc_mesh():
    return plsc.VectorSubcoreMesh(core_axis_name="c", subcore_axis_name="s")


def _iota16():
    return lax.iota(jnp.int32, 16)


def _full16(v):
    return jnp.full((16,), v, jnp.int32)


def _proj_body(nrel, x_ref, wq_ref, wk_ref, wv_ref, bq_ref, bk_ref, bv_ref,
               a_ref, m_ref, scl_ref, o_ref):
    x = x_ref[...]
    q = jnp.dot(x, wq_ref[...], preferred_element_type=jnp.float32) + bq_ref[...]
    k = jnp.dot(x, wk_ref[...], preferred_element_type=jnp.float32) + bk_ref[...]
    v = jnp.dot(x, wv_ref[...], preferred_element_type=jnp.float32) + bv_ref[...]
    o_ref[0] = q
    for r in range(nrel):
        kcols = []
        vcols = []
        for h in range(H):
            kh = jnp.dot(k[:, h * D:(h + 1) * D], a_ref[r, h],
                         preferred_element_type=jnp.float32)
            kcols.append(kh * scl_ref[0, r * H + h])
            vcols.append(jnp.dot(v[:, h * D:(h + 1) * D], m_ref[r, h],
                                 preferred_element_type=jnp.float32))
        o_ref[1 + 2 * r] = jnp.concatenate(kcols, axis=1)
        o_ref[2 + 2 * r] = jnp.concatenate(vcols, axis=1)


def _proj(x, wq, wk, wv, bq, bk, bv, a_stack, m_stack, scl):
    n = x.shape[0]
    nrel = a_stack.shape[0]
    nproj = 1 + 2 * nrel
    grid = n // NB
    full_w = pl.BlockSpec((C, C), lambda i: (0, 0))
    full_b = pl.BlockSpec((1, C), lambda i: (0, 0))
    return pl.pallas_call(
        functools.partial(_proj_body, nrel),
        grid=(grid,),
        in_specs=[
            pl.BlockSpec((NB, C), lambda i: (i, 0)),
            full_w, full_w, full_w, full_b, full_b, full_b,
            pl.BlockSpec((nrel, H, D, D), lambda i: (0, 0, 0, 0)),
            pl.BlockSpec((nrel, H, D, D), lambda i: (0, 0, 0, 0)),
            pl.BlockSpec(memory_space=pltpu.SMEM),
        ],
        out_specs=pl.BlockSpec((nproj, NB, C), lambda i: (0, i, 0)),
        out_shape=jax.ShapeDtypeStruct((nproj, n, C), jnp.float32),
    )(x, wq, wk, wv, bq.reshape(1, C), bk.reshape(1, C), bv.reshape(1, C),
      a_stack, m_stack, scl)


def _logits_kernel(rels, args, outs, scr):
    cid = lax.axis_index("c")
    sid = lax.axis_index("s")
    wid = sid * NCORES + cid
    srcv, dstv, qg, kg, lb, mxe, mxp, sem = scr
    mxe[...] = jnp.full((16,), NEG, jnp.float32)
    mxp[...] = jnp.full((16,), NEG, jnp.float32)

    for r in rels:
        qtbl = args[r["qtbl"]]
        ktbl = args[r["ktbl"]]
        src = args[r["src"]]
        dst = args[r["dst"]]
        mxref = mxe if r["dst_is_entity"] else mxp
        epw = r["ep"] // NW
        nc = epw // CH
        e_real = r["e_real"]

        def chunk(ci, _):
            base = wid * epw + ci * CH
            c1 = pltpu.async_copy(src.at[pl.ds(base, CH)], srcv, sem)
            c2 = pltpu.async_copy(dst.at[pl.ds(base, CH)], dstv, sem)
            c1.wait()
            c2.wait()
            g1 = pltpu.async_copy(qtbl.at[0].at[dstv], qg, sem)
            g2 = pltpu.async_copy(ktbl.at[r["kj"]].at[srcv], kg, sem)
            g1.wait()
            g2.wait()
            for h in range(H):

                def grp(g, _):
                    rows = _iota16() + g * 16
                    acc = jnp.zeros((16,), jnp.float32)
                    for c in range(D):
                        qc = plsc.load_gather(qg, [rows, _full16(h * D + c)])
                        kc = plsc.load_gather(kg, [rows, _full16(h * D + c)])
                        acc = acc + qc * kc
                    gid = base + rows
                    acc = jnp.where(gid < e_real, acc,
                                    jnp.full((16,), NEG, jnp.float32))
                    lb[pl.ds(g * 16, 16)] = acc
                    mxref[...] = jnp.maximum(mxref[...], acc)
                    return 0

                lax.fori_loop(0, CH // 16, grp, 0)
                pltpu.sync_copy(lb, outs[r["lg0"] + h].at[pl.ds(base, CH)])
            return 0

        lax.fori_loop(0, nc, chunk, 0)

    pltpu.sync_copy(mxe, outs[-2].at[pl.ds(wid * 16, 16)])
    pltpu.sync_copy(mxp, outs[-1].at[pl.ds(wid * 16, 16)])


def _run_logits(tbl_e, tbl_p, ee, pe, ep):
    rels = [
        dict(qtbl=0, ktbl=0, kj=1, src=2, dst=3, dst_is_entity=True,
             ep=ee["ep_pad"], e_real=ee["e_real"], lg0=0),
        dict(qtbl=0, ktbl=1, kj=1, src=4, dst=5, dst_is_entity=True,
             ep=pe["ep_pad"], e_real=pe["e_real"], lg0=4),
        dict(qtbl=1, ktbl=0, kj=3, src=6, dst=7, dst_is_entity=False,
             ep=ep["ep_pad"], e_real=ep["e_real"], lg0=8),
    ]
    out_type = ([jax.ShapeDtypeStruct((ee["ep_pad"],), jnp.float32)] * 4
                + [jax.ShapeDtypeStruct((pe["ep_pad"],), jnp.float32)] * 4
                + [jax.ShapeDtypeStruct((ep["ep_pad"],), jnp.float32)] * 4
                + [jax.ShapeDtypeStruct((NW * 16,), jnp.float32)] * 2)

    @functools.partial(
        pl.kernel, mesh=_sc_mesh(), out_type=out_type,
        compiler_params=pltpu.CompilerParams(needs_layout_passes=False),
        scratch_types=[
            pltpu.VMEM((CH,), jnp.int32),
            pltpu.VMEM((CH,), jnp.int32),
            pltpu.VMEM((CH, C), jnp.float32),
            pltpu.VMEM((CH, C), jnp.float32),
            pltpu.VMEM((CH,), jnp.float32),
            pltpu.VMEM((16,), jnp.float32),
            pltpu.VMEM((16,), jnp.float32),
            pltpu.SemaphoreType.DMA,
        ],
    )
    def k(te, tp, ees, eed, pes, ped, eps, epd,
          l0, l1, l2, l3, l4, l5, l6, l7, l8, l9, l10, l11, me, mp,
          srcv, dstv, qg, kg, lb, mxe, mxp, sem):
        _logits_kernel(rels, [te, tp, ees, eed, pes, ped, eps, epd],
                       [l0, l1, l2, l3, l4, l5, l6, l7, l8, l9, l10, l11,
                        me, mp],
                       [srcv, dstv, qg, kg, lb, mxe, mxp, sem])

    res = k(tbl_e, tbl_p, ee["src"], ee["dst"], pe["src"], pe["dst"],
            ep["src"], ep["dst"])
    return res[0:4], res[4:8], res[8:12], res[12], res[13]


def _accum_kernel(nseg, nsweeps, packed_den, rels, args, outs, scr):
    cid = lax.axis_index("c")
    sid = lax.axis_index("s")
    outn, outd = outs
    srcv, dstv, vg, stage, lb0, lb1, lb2, lb3, mxall, zbuf, acc, sem = scr
    lbs = [lb0, lb1, lb2, lb3]
    qrange = nseg // (NCORES * nsweeps)
    rpt = qrange // NSUB

    pltpu.sync_copy(args[0], mxall)
    m = jnp.full((16,), NEG, jnp.float32)
    for i in range(NW):
        m = jnp.maximum(m, mxall[pl.ds(i * 16, 16)])
    mv = jnp.full((16,), jnp.max(m), jnp.float32)

    def zrow(i, _):
        for c in range(C // 16):
            zbuf[i, pl.ds(c * 16, 16)] = jnp.zeros((16,), jnp.float32)
        return 0
    lax.fori_loop(0, ZR, zrow, 0)

    if packed_den:
        wid32 = sid * NCORES + cid
        rptd = (nseg // 4) // NSUB
        for j in range(rptd // ZR):
            pltpu.sync_copy(zbuf, acc.at[pl.ds(sid * rptd + j * ZR, ZR)])
        plsc.subcore_barrier()

        def sclr(i, _):
            for cb in range(C // 16):
                stage[i, pl.ds(cb * 16, 16)] = jnp.zeros((16,), jnp.float32)
            return 0
        lax.fori_loop(0, CHB, sclr, 0)

        for r in rels:
            dst = args[r["dst"]]
            lgs = [args[r["lg0"] + h] for h in range(H)]
            epw = r["ep"] // NW
            nc = epw // CHB

            def chunk_d(ci, _):
                base = wid32 * epw + ci * CHB
                cps = [pltpu.async_copy(dst.at[pl.ds(base, CHB)], dstv, sem)]
                for h in range(H):
                    cps.append(pltpu.async_copy(
                        lgs[h].at[pl.ds(base, CHB)], lbs[h], sem))
                for cp in cps:
                    cp.wait()

                def grp(g, _):
                    rows = _iota16() + g * 16
                    dl = dstv[pl.ds(g * 16, 16)]
                    dstv[pl.ds(g * 16, 16)] = lax.shift_right_logical(
                        dl, jnp.full((16,), 2, jnp.int32))
                    sub = (dl & jnp.full((16,), 3, jnp.int32)) * 4
                    for k in range(16):
                        plsc.store_scatter(stage, [rows, _full16(k)],
                                           jnp.zeros((16,), jnp.float32))
                    for h in range(H):
                        w = jnp.exp(lbs[h][pl.ds(g * 16, 16)] - mv)
                        plsc.store_scatter(stage, [rows, sub + h], w)
                    return 0

                lax.fori_loop(0, CHB // 16, grp, 0)
                pltpu.sync_copy(stage, acc.at[dstv], add=True)
                return 0

            lax.fori_loop(0, nc, chunk_d, 0)
        plsc.subcore_barrier()
        offd = pl.multiple_of(sid * rptd, 8)
        pltpu.sync_copy(acc.at[pl.ds(sid * rptd, rptd)],
                        outd.at[cid, pl.ds(offd, rptd)])
        plsc.subcore_barrier()
        mode_list = ((0, outn),)
    else:
        mode_list = ((0, outn), (1, outd))

    for s in range(nsweeps):
        row0 = (s * NCORES + cid) * qrange
        for mode, out in mode_list:
            for j in range(rpt // ZR):
                pltpu.sync_copy(zbuf, acc.at[pl.ds(sid * rpt + j * ZR, ZR)])
            plsc.subcore_barrier()

            for r in rels:
                vtbl = args[r["vtbl"]]
                src = args[r["src"]]
                dst = args[r["dst"]]
                lgs = [args[r["lg0"] + h] for h in range(H)]
                epw = r["ep"] // NSUB
                nc = epw // CHB

                def chunk(ci, _):
                    base = sid * epw + ci * CHB
                    cps = [pltpu.async_copy(dst.at[pl.ds(base, CHB)], dstv,
                                            sem)]
                    for h in range(H):
                        cps.append(pltpu.async_copy(
                            lgs[h].at[pl.ds(base, CHB)], lbs[h], sem))
                    if mode == 0:
                        cps.append(pltpu.async_copy(
                            src.at[pl.ds(base, CHB)], srcv, sem))
                    for cp in cps:
                        cp.wait()
                    if mode == 0:
                        pltpu.async_copy(vtbl.at[r["vj"]].at[srcv], vg,
                                         sem).wait()

                    def grp(g, _):
                        rows = _iota16() + g * 16
                        dl = dstv[pl.ds(g * 16, 16)] - row0
                        inr = (dl >= 0) & (dl < qrange)
                        dstv[pl.ds(g * 16, 16)] = jnp.where(
                            inr, dl, jnp.zeros((16,), jnp.int32))
                        for h in range(H):
                            w = jnp.exp(lbs[h][pl.ds(g * 16, 16)] - mv)
                            w = jnp.where(inr, w,
                                          jnp.zeros((16,), jnp.float32))
                            for c in range(D):
                                col = _full16(h * D + c)
                                if mode == 0:
                                    vc = plsc.load_gather(vg, [rows, col])
                                    plsc.store_scatter(stage, [rows, col],
                                                       vc * w)
                                else:
                                    plsc.store_scatter(stage, [rows, col], w)
                        return 0

                    lax.fori_loop(0, CHB // 16, grp, 0)
                    pltpu.sync_copy(stage, acc.at[dstv], add=True)
                    return 0

                lax.fori_loop(0, nc, chunk, 0)

            plsc.subcore_barrier()
            off = pl.multiple_of(row0 + sid * rpt, 8)
            pltpu.sync_copy(acc.at[pl.ds(sid * rpt, rpt)],
                            out.at[pl.ds(off, rpt)])
            plsc.subcore_barrier()


def _run_accum(nseg, nsweeps, packed_den, rels_cfg, arrays):
    qrange = nseg // (NCORES * nsweeps)
    if packed_den:
        outd_t = jax.ShapeDtypeStruct((NCORES, nseg // 4, C), jnp.float32)
    else:
        outd_t = jax.ShapeDtypeStruct((nseg, C), jnp.float32)

    @functools.partial(
        pl.kernel, mesh=_sc_mesh(),
        out_type=[jax.ShapeDtypeStruct((nseg, C), jnp.float32), outd_t],
        compiler_params=pltpu.CompilerParams(needs_layout_passes=False),
        scratch_types=[
            pltpu.VMEM((CHB,), jnp.int32),
            pltpu.VMEM((CHB,), jnp.int32),
            pltpu.VMEM((CHB, C), jnp.float32),
            pltpu.VMEM((CHB, C), jnp.float32),
            pltpu.VMEM((CHB,), jnp.float32),
            pltpu.VMEM((CHB,), jnp.float32),
            pltpu.VMEM((CHB,), jnp.float32),
            pltpu.VMEM((CHB,), jnp.float32),
            pltpu.VMEM((NW * 16,), jnp.float32),
            pltpu.VMEM((ZR, C), jnp.float32),
            pltpu.VMEM_SHARED((qrange, C), jnp.float32),
            pltpu.SemaphoreType.DMA,
        ],
    )
    def k(*refs):
        nargs = len(arrays)
        args = refs[:nargs]
        outs = refs[nargs:nargs + 2]
        scr = refs[nargs + 2:]
        _accum_kernel(nseg, nsweeps, packed_den, rels_cfg, args, outs, scr)

    return k(*arrays)


def _gelu(x):
    return 0.5 * x * (1.0 + lax.erf(x * 0.7071067811865476))


def _finish_body(num_ref, den_ref, x_ref, w_ref, b_ref, beta_ref, o_ref):
    num = num_ref[...]
    den = den_ref[...]
    cols = []
    for h in range(H):
        cols.append(num[:, h * D:(h + 1) * D]
                    / (den[:, h * D:h * D + 1] + 1e-16))
    agg = jnp.concatenate(cols, axis=1)
    out = jnp.dot(_gelu(agg), w_ref[...],
                  preferred_element_type=jnp.float32) + b_ref[...]
    beta = beta_ref[0, 0]
    o_ref[...] = _gelu(beta * out + (1.0 - beta) * x_ref[...])


NBE = 160


def _finish_e_body(num_ref, den_ref, x_ref, w_ref, b_ref, beta_ref, o_ref):
    num = num_ref[...]
    dsum = den_ref[0] + den_ref[1]
    den_rep = jnp.broadcast_to(dsum[:, None, :],
                               (NBE // 4, 4, C)).reshape(NBE, C)
    rowmod = lax.broadcasted_iota(jnp.int32, (NBE, 1), 0) % 4
    cols = []
    for h in range(H):
        den_h = jnp.zeros((NBE, 1), jnp.float32)
        for j in range(4):
            den_h += (den_rep[:, j * 4 + h:j * 4 + h + 1]
                      * (rowmod == j).astype(jnp.float32))
        cols.append(num[:, h * D:(h + 1) * D] / (den_h + 1e-16))
    agg = jnp.concatenate(cols, axis=1)
    out = jnp.dot(_gelu(agg), w_ref[...],
                  preferred_element_type=jnp.float32) + b_ref[...]
    beta = beta_ref[0, 0]
    o_ref[...] = _gelu(beta * out + (1.0 - beta) * x_ref[...])


def _finish_e(num, den, x, w, b, beta):
    n = x.shape[0]
    grid = n // NBE
    return pl.pallas_call(
        _finish_e_body,
        grid=(grid,),
        in_specs=[
            pl.BlockSpec((NBE, C), lambda i: (i, 0)),
            pl.BlockSpec((NCORES, NBE // 4, C), lambda i: (0, i, 0)),
            pl.BlockSpec((NBE, C), lambda i: (i, 0)),
            pl.BlockSpec((C, C), lambda i: (0, 0)),
            pl.BlockSpec((1, C), lambda i: (0, 0)),
            pl.BlockSpec(memory_space=pltpu.SMEM),
        ],
        out_specs=pl.BlockSpec((NBE, C), lambda i: (i, 0)),
        out_shape=jax.ShapeDtypeStruct((n, C), jnp.float32),
    )(num, den, x, w, b.reshape(1, C), beta)


def _finish(num, den, x, w, b, beta):
    n = x.shape[0]
    grid = n // NB
    return pl.pallas_call(
        _finish_body,
        grid=(grid,),
        in_specs=[
            pl.BlockSpec((NB, C), lambda i: (i, 0)),
            pl.BlockSpec((NB, C), lambda i: (i, 0)),
            pl.BlockSpec((NB, C), lambda i: (i, 0)),
            pl.BlockSpec((C, C), lambda i: (0, 0)),
            pl.BlockSpec((1, C), lambda i: (0, 0)),
            pl.BlockSpec(memory_space=pltpu.SMEM),
        ],
        out_specs=pl.BlockSpec((NB, C), lambda i: (i, 0)),
        out_shape=jax.ShapeDtypeStruct((n, C), jnp.float32),
    )(num, den, x, w, b.reshape(1, C), beta)


def _ln_rows(x, g, b):
    m = jnp.mean(x, axis=1, keepdims=True)
    v = jnp.mean((x - m) ** 2, axis=1, keepdims=True)
    return (x - m) * lax.rsqrt(v + 1e-5) * g + b


def _sg1_body(he_ref, fcw_ref, fcb_ref, lng_ref, lnb_ref,
              wq_ref, bq_ref, wk_ref, bk_ref, wv_ref, bv_ref,
              x0_ref, kvs_ref, ks_ref, ss_ref):
    pi = pl.program_id(0)
    x0 = jnp.maximum(
        _ln_rows(jnp.dot(he_ref[...], fcw_ref[...],
                         preferred_element_type=jnp.float32) + fcb_ref[...],
                 lng_ref[...], lnb_ref[...]), 0.0)
    x0_ref[...] = x0
    qu = jnp.dot(x0, wq_ref[...], preferred_element_type=jnp.float32) + bq_ref[...]
    ku = jnp.dot(x0, wk_ref[...], preferred_element_type=jnp.float32) + bk_ref[...]
    vu = jnp.dot(x0, wv_ref[...], preferred_element_type=jnp.float32) + bv_ref[...]

    @pl.when(pi == 0)
    def _():
        kvs_ref[...] = jnp.zeros_like(kvs_ref)
        ks_ref[...] = jnp.zeros_like(ks_ref)
        ss_ref[...] = jnp.zeros_like(ss_ref)

    ksums = []
    for h in range(H):
        kh = ku[:, h * C:(h + 1) * C]
        vh = vu[:, h * C:(h + 1) * C]
        kvs_ref[h] += lax.dot_general(
            kh, vh, (((0,), (0,)), ((), ())),
            preferred_element_type=jnp.float32)
        ksums.append(jnp.sum(kh, axis=0, keepdims=True))
    ks_ref[...] += jnp.stack(ksums, axis=1)
    ss_ref[...] += jnp.concatenate(
        [jnp.full((1, 64), jnp.sum(qu * qu), jnp.float32),
         jnp.full((1, 64), jnp.sum(ku * ku), jnp.float32)], axis=1)


def _sg2_body(he_ref, x0_ref, kvs_ref, ks_ref, ss_ref,
              wq_ref, bq_ref, wv_ref, bv_ref,
              ln1g_ref, ln1b_ref, ng_ref, nb_ref,
              w1_ref, b1_ref, w2_ref, b2_ref, o_ref):
    x0 = x0_ref[...]
    qu = jnp.dot(x0, wq_ref[...], preferred_element_type=jnp.float32) + bq_ref[...]
    vu = jnp.dot(x0, wv_ref[...], preferred_element_type=jnp.float32) + bv_ref[...]
    ss = ss_ref[...]
    nq = jnp.sqrt(ss[0, 0])
    nk = jnp.sqrt(ss[0, 64])
    ks = ks_ref[...]
    n_nodes = jnp.float32(NE)
    acc = jnp.zeros((NB, C), jnp.float32)
    for h in range(H):
        qh = qu[:, h * C:(h + 1) * C] / nq
        vh = vu[:, h * C:(h + 1) * C]
        num = jnp.dot(qh, kvs_ref[h] / nk,
                      preferred_element_type=jnp.float32) + n_nodes * vh
        ks_row = ks[0, h, :].reshape(1, C) / nk
        den = jnp.sum(qh * ks_row, axis=1, keepdims=True) + n_nodes
        acc += num / den
    out = acc / jnp.float32(H)
    hg = _ln_rows(0.5 * out + 0.5 * x0, ln1g_ref[...], ln1b_ref[...])
    hf = _ln_rows(0.9 * he_ref[...] + 0.1 * hg, ng_ref[...], nb_ref[...])
    z = jnp.maximum(jnp.dot(hf, w1_ref[...],
                            preferred_element_type=jnp.float32) + b1_ref[...],
                    0.0)
    o_ref[...] = jnp.dot(z, w2_ref[...],
                         preferred_element_type=jnp.float32) + b2_ref[...]


def _mlp_body(x_ref, w1_ref, b1_ref, w2_ref, b2_ref, o_ref):
    z = jnp.maximum(jnp.dot(x_ref[...], w1_ref[...],
                            preferred_element_type=jnp.float32) + b1_ref[...],
                    0.0)
    o_ref[...] = jnp.dot(z, w2_ref[...],
                         preferred_element_type=jnp.float32) + b2_ref[...]


def _row_spec():
    return pl.BlockSpec((NB, C), lambda i: (i, 0))


def _w_spec(r, c):
    return pl.BlockSpec((r, c), lambda i: (0, 0))


def _b_spec(c):
    return pl.BlockSpec((1, c), lambda i: (0, 0))


def _sgformer_and_heads(he, sg, norm_g, norm_b, pe_w1, pe_b1, pe_w2, pe_b2):
    grid = NE // NB
    x0, kvs, ks, ss = pl.pallas_call(
        _sg1_body,
        grid=(grid,),
        in_specs=[
            _row_spec(), _w_spec(C, C), _b_spec(C), _b_spec(C), _b_spec(C),
            _w_spec(C, C * H), _b_spec(C * H),
            _w_spec(C, C * H), _b_spec(C * H),
            _w_spec(C, C * H), _b_spec(C * H),
        ],
        out_specs=[
            pl.BlockSpec((NB, C), lambda i: (i, 0)),
            pl.BlockSpec((H, C, C), lambda i: (0, 0, 0)),
            pl.BlockSpec((1, H, C), lambda i: (0, 0, 0)),
            pl.BlockSpec((1, C), lambda i: (0, 0)),
        ],
        out_shape=[
            jax.ShapeDtypeStruct((NE, C), jnp.float32),
            jax.ShapeDtypeStruct((H, C, C), jnp.float32),
            jax.ShapeDtypeStruct((1, H, C), jnp.float32),
            jax.ShapeDtypeStruct((1, C), jnp.float32),
        ],
    )(he, sg['fc0_W'], sg['fc0_b'].reshape(1, C),
      sg['ln0_g'].reshape(1, C), sg['ln0_b'].reshape(1, C),
      sg['Wq'], sg['bq'].reshape(1, C * H),
      sg['Wk'], sg['bk'].reshape(1, C * H),
      sg['Wv'], sg['bv'].reshape(1, C * H))

    z_e = pl.pallas_call(
        _sg2_body,
        grid=(grid,),
        in_specs=[
            _row_spec(), _row_spec(),
            pl.BlockSpec((H, C, C), lambda i: (0, 0, 0)),
            pl.BlockSpec((1, H, C), lambda i: (0, 0, 0)),
            _b_spec(C),
            _w_spec(C, C * H), _b_spec(C * H),
            _w_spec(C, C * H), _b_spec(C * H),
            _b_spec(C), _b_spec(C), _b_spec(C), _b_spec(C),
            _w_spec(C, C), _b_spec(C), _w_spec(C, C), _b_spec(C),
        ],
        out_specs=pl.BlockSpec((NB, C), lambda i: (i, 0)),
        out_shape=jax.ShapeDtypeStruct((NE, C), jnp.float32),
    )(he, x0, kvs, ks, ss,
      sg['Wq'], sg['bq'].reshape(1, C * H),
      sg['Wv'], sg['bv'].reshape(1, C * H),
      sg['ln1_g'].reshape(1, C), sg['ln1_b'].reshape(1, C),
      norm_g.reshape(1, C), norm_b.reshape(1, C),
      pe_w1, pe_b1.reshape(1, C), pe_w2, pe_b2.reshape(1, C))
    return z_e


def _mlp_call(x, w1, b1, w2, b2):
    n = x.shape[0]
    return pl.pallas_call(
        _mlp_body,
        grid=(n // NB,),
        in_specs=[_row_spec(), _w_spec(C, C), _b_spec(C),
                  _w_spec(C, C), _b_spec(C)],
        out_specs=pl.BlockSpec((NB, C), lambda i: (i, 0)),
        out_shape=jax.ShapeDtypeStruct((n, C), jnp.float32),
    )(x, w1, b1.reshape(1, C), w2, b2.reshape(1, C))


def _pad_edges(ei):
    e = ei.shape[1]
    ep = ((e + ALIGN - 1) // ALIGN) * ALIGN
    pad = ep - e
    src = jnp.concatenate(
        [ei[0].astype(jnp.int32), jnp.zeros((pad,), jnp.int32)])
    dst = jnp.concatenate(
        [ei[1].astype(jnp.int32), jnp.zeros((pad,), jnp.int32)])
    return dict(src=src, dst=dst, ep_pad=ep, e_real=e)


def _hgt_layer(xe, xp, ee, pe, ep, p):
    sd = float(D) ** 0.5
    scl_e = jnp.concatenate([p['p_rel'][0] / sd, p['p_rel'][1] / sd]).reshape(1, 2 * H)
    scl_p = (p['p_rel'][2] / sd).reshape(1, H)
    tbl_e = _proj(xe, p['Wq'][0], p['Wk'][0], p['Wv'][0],
                  p['bq'][0], p['bk'][0], p['bv'][0],
                  jnp.stack([p['a_rel'][0], p['a_rel'][1]]),
                  jnp.stack([p['m_rel'][0], p['m_rel'][1]]), scl_e)
    tbl_p = _proj(xp, p['Wq'][1], p['Wk'][1], p['Wv'][1],
                  p['bq'][1], p['bk'][1], p['bv'][1],
                  p['a_rel'][2][None], p['m_rel'][2][None], scl_p)

    lg_ee, lg_pe, lg_ep, mx_e, mx_p = _run_logits(tbl_e, tbl_p, ee, pe, ep)

    rels_e = [
        dict(vtbl=1, vj=2, src=3, dst=4, lg0=5, ep=ee["ep_pad"]),
        dict(vtbl=2, vj=2, src=9, dst=10, lg0=11, ep=pe["ep_pad"]),
    ]
    args_e = ([mx_e, tbl_e, tbl_p, ee["src"], ee["dst"]] + list(lg_ee)
              + [pe["src"], pe["dst"]] + list(lg_pe))
    num_e, den_e = _run_accum(NE_PAD, 2, True, rels_e, args_e)

    rels_p = [dict(vtbl=1, vj=4, src=2, dst=3, lg0=4, ep=ep["ep_pad"])]
    args_p = [mx_p, tbl_e, ep["src"], ep["dst"]] + list(lg_ep)
    num_p, den_p = _run_accum(NP_PAD, 1, False, rels_p, args_p)

    be = jax.nn.sigmoid(p['skip'][0]).reshape(1, 1)
    bp = jax.nn.sigmoid(p['skip'][1]).reshape(1, 1)
    he = _finish_e(num_e, den_e, xe, p['Wout'][0], p['bout'][0], be)
    hp = _finish(num_p, den_p, xp, p['Wout'][1], p['bout'][1], bp)
    return he, hp


def kernel(x_entity, x_passage, params, ei_ee, ei_ep, ei_pe):
    ee = _pad_edges(ei_ee)
    ep = _pad_edges(ei_ep)
    pe = _pad_edges(ei_pe)
    he, hp = x_entity, x_passage
    for lp in params['hgt']:
        he, hp = _hgt_layer(he, hp, ee, pe, ep, lp)
    z_e = _sgformer_and_heads(he, params['sg'], params['norm_g'],
                              params['norm_b'], params['pe_W1'],
                              params['pe_b1'], params['pe_W2'],
                              params['pe_b2'])
    z_p = _mlp_call(hp, params['pp_W1'], params['pp_b1'],
                    params['pp_W2'], params['pp_b2'])
    return z_e, z_p
